# Initial kernel scaffold; baseline (speedup 1.0000x reference)
import jax
import jax.numpy as jnp
from jax.experimental import pallas as pl


def kernel(x, adj_indices, adj_values):
    raise NotImplementedError("write your pallas kernel here")


# speedup vs baseline: 4.1374x; 4.1374x over previous
import dataclasses
import functools

import jax
import jax.numpy as jnp
from jax import lax
from jax.experimental import pallas as pl
from jax.experimental.pallas import tpu as pltpu
from jax.experimental.pallas import tpu_sc as plsc

NC = 2
NS = 16
NW = NC * NS
LANES = 16
K = 128


def _sc_partials(x, src, dst, val, n_chunks, n_nodes, d_feat):
  n_pad = -(-n_nodes // (NS * K)) * NS * K
  rows_per_tile = n_pad // NS
  zrows = K
  n_zero_copies = rows_per_tile // zrows
  mesh = plsc.VectorSubcoreMesh(
      core_axis_name="c", subcore_axis_name="s", num_cores=NC,
      num_subcores=NS)
  cp = pltpu.CompilerParams()
  if "needs_layout_passes" in pltpu.CompilerParams.__dataclass_fields__:
    cp = dataclasses.replace(cp, needs_layout_passes=False)

  @functools.partial(
      pl.kernel,
      compiler_params=cp,
      out_type=jax.ShapeDtypeStruct((NC, n_pad, d_feat), jnp.float32),
      mesh=mesh,
      scratch_types=[
          pltpu.VMEM_SHARED((n_pad, d_feat), jnp.float32),
          pltpu.VMEM((n_chunks, K), jnp.int32),
          pltpu.VMEM((n_chunks, K), jnp.int32),
          pltpu.VMEM((n_chunks, K), jnp.float32),
          pltpu.VMEM((K, d_feat), jnp.float32),
      ],
  )
  def sc_kernel(x_hbm, src_hbm, dst_hbm, val_hbm, part_hbm,
                acc, src_v, dst_v, val_v, rows):
    c = lax.axis_index("c")
    s = lax.axis_index("s")
    wid = s * NC + c

    pltpu.sync_copy(src_hbm.at[wid], src_v)
    pltpu.sync_copy(dst_hbm.at[wid], dst_v)
    pltpu.sync_copy(val_hbm.at[wid], val_v)

    @pl.loop(0, K)
    def _(i):
      for t in range(d_feat // LANES):
        rows.at[i, pl.ds(t * LANES, LANES)][...] = jnp.zeros(
            (LANES,), jnp.float32)

    for k in range(n_zero_copies):
      pltpu.sync_copy(
          rows.at[pl.ds(0, zrows)],
          acc.at[pl.ds(s * rows_per_tile + k * zrows, zrows)])
    plsc.subcore_barrier()

    @pl.loop(0, n_chunks)
    def _(j):
      pltpu.sync_copy(x_hbm.at[src_v.at[j]], rows)

      @pl.loop(0, K)
      def _(e):
        vsplat = plsc.load_gather(
            val_v,
            [jnp.full((LANES,), j, jnp.int32),
             jnp.full((LANES,), e, jnp.int32)])
        for t in range(d_feat // LANES):
          sl = rows.at[e, pl.ds(t * LANES, LANES)]
          sl[...] = sl[...] * vsplat

      pltpu.sync_copy(rows, acc.at[dst_v.at[j]], add=True)

    plsc.subcore_barrier()

    for k in range(n_zero_copies):
      r0 = s * rows_per_tile + k * zrows
      pltpu.sync_copy(acc.at[pl.ds(r0, zrows)],
                      part_hbm.at[c, pl.ds(r0, zrows)])

  return sc_kernel(x, src, dst, val)


def _tc_merge(partials, n_nodes, d_feat):
  def body(p_ref, o_ref):
    o_ref[...] = p_ref[0, :n_nodes] + p_ref[1, :n_nodes]

  return pl.pallas_call(
      body,
      out_shape=jax.ShapeDtypeStruct((n_nodes, d_feat), jnp.float32),
  )(partials)


@jax.jit
def _run(x, adj_indices, adj_values):
  n_nodes, d_feat = x.shape
  n_edges = adj_values.shape[0]
  dst = adj_indices[0].astype(jnp.int32)
  src = adj_indices[1].astype(jnp.int32)
  val = adj_values.astype(jnp.float32)

  n_chunks = -(-n_edges // (NW * K))
  e_pad = NW * n_chunks * K
  pad = e_pad - n_edges
  src = jnp.pad(src, (0, pad)).reshape(NW, n_chunks, K)
  dst = jnp.pad(dst, (0, pad)).reshape(NW, n_chunks, K)
  val = jnp.pad(val, (0, pad)).reshape(NW, n_chunks, K)

  partials = _sc_partials(x, src, dst, val, n_chunks, n_nodes, d_feat)
  return _tc_merge(partials, n_nodes, d_feat)


def kernel(x, adj_indices, adj_values):
  return _run(x, adj_indices, adj_values)
